# baseline (speedup 1.0000x reference)
import functools
import numpy as np
import jax
import jax.numpy as jnp
from jax import lax
from jax.experimental import pallas as pl
from jax.experimental.pallas import tpu as pltpu
from jax.experimental.pallas import tpu_sc as plsc

_N_NODES = 10000
_N_EDGES = 320000
_NU = 64
_NB = 16

_NC = 2
_NS = 16
_NW = _NC * _NS
_CHUNK = 256
_NCHUNKS = _N_EDGES // _CHUNK
_NT_HI = (_NCHUNKS + _NW - 1) // _NW
_NT_LO = _NCHUNKS // _NW
_NREM = _NCHUNKS - _NT_LO * _NW
_N_PAD = 10240
_ROWS_PER_TILE = _N_PAD // _NS

_BLK = 1000
_GRID = _N_NODES // _BLK

_SGN = np.ones((1, 64), np.float32)
for _i in range(16):
    _SGN[0, 3 * _i + 1] = -1.0
_MASK = np.zeros((1, 64), np.float32)
_MASK[0, 48:] = -1e30
_G = np.zeros((64, 64), np.float32)
for _i in range(16):
    _G[3 * _i:3 * _i + 3, 3 * _i:3 * _i + 3] = 1.0
_ESELN = np.zeros((64, 32), np.float32)
_OSEL = np.zeros((64, 32), np.float32)
_PE = np.zeros((32, 64), np.float32)
_PO = np.zeros((32, 64), np.float32)
_PADW = np.zeros((16, 128), np.float32)
for _j in range(16):
    _PADW[_j, _j] = 1.0
for _j in range(32):
    _ESELN[2 * _j, _j] = -1.0
    _OSEL[2 * _j + 1, _j] = 1.0
    _PE[_j, 2 * _j] = 1.0
    _PO[_j, 2 * _j + 1] = 1.0


def _unary_body(x_ref, w_ref, sgn_ref, msk_ref, g_ref, en_ref, os_ref,
                u_ref, a_ref, b_ref):
    x = x_ref[...]
    cm = x * sgn_ref[...] + msk_ref[...]
    m = jnp.max(cm, axis=1, keepdims=True)
    e = jnp.exp(cm - m)
    s = jnp.dot(e, g_ref[...], preferred_element_type=jnp.float32)
    r = e / (s + 1e-30)
    u = x + sgn_ref[...] * r * w_ref[...]
    u_ref[...] = u
    a_ref[...] = jnp.exp(jnp.dot(u, en_ref[...], preferred_element_type=jnp.float32))
    b_ref[...] = jnp.exp(jnp.dot(u, os_ref[...], preferred_element_type=jnp.float32))


def _stage_a(unary, wcol):
    const_spec = lambda shape: pl.BlockSpec(shape, lambda i: (0, 0))
    return pl.pallas_call(
        _unary_body,
        grid=(_GRID,),
        in_specs=[
            pl.BlockSpec((_BLK, 64), lambda i: (i, 0)),
            const_spec((1, 64)), const_spec((1, 64)), const_spec((1, 64)),
            const_spec((64, 64)), const_spec((64, 32)), const_spec((64, 32)),
        ],
        out_specs=[
            pl.BlockSpec((_BLK, 64), lambda i: (i, 0)),
            pl.BlockSpec((_BLK, 32), lambda i: (i, 0)),
            pl.BlockSpec((_BLK, 32), lambda i: (i, 0)),
        ],
        out_shape=[
            jax.ShapeDtypeStruct((_N_NODES, 64), jnp.float32),
            jax.ShapeDtypeStruct((_N_NODES, 32), jnp.float32),
            jax.ShapeDtypeStruct((_N_NODES, 32), jnp.float32),
        ],
    )(unary, wcol, jnp.asarray(_SGN), jnp.asarray(_MASK), jnp.asarray(_G),
      jnp.asarray(_ESELN), jnp.asarray(_OSEL))


def _sc_body(a_hbm, b_hbm, bin_hbm, i1_hbm, i2_hbm, w_hbm,
             acce_hbm, acco_hbm, bout_hbm,
             i1_v, i2_v, i1all, i2all, ga, gb, bin_v, w_v, zbuf,
             acce_s, acco_s,
             sga, sgb, sbi, sca, scb, sbo):
    cid = lax.axis_index("c")
    sid = lax.axis_index("s")
    wid = sid * _NC + cid

    zeros16 = jnp.zeros((16,), jnp.float32)

    def zrow(i, _):
        zbuf[i, 0:16] = zeros16
        zbuf[i, 16:32] = zeros16
        return 0

    lax.fori_loop(0, _ROWS_PER_TILE, zrow, 0)
    pltpu.sync_copy(zbuf, acce_s.at[pl.ds(sid * _ROWS_PER_TILE, _ROWS_PER_TILE)])
    pltpu.sync_copy(zbuf, acco_s.at[pl.ds(sid * _ROWS_PER_TILE, _ROWS_PER_TILE)])
    pltpu.sync_copy(w_hbm, w_v)
    plsc.subcore_barrier()

    wlo = w_v[0:16]
    whi = w_v[16:32]

    n_t = jnp.where(wid < _NREM, _NT_HI, _NT_LO)
    start = jnp.where(wid < _NREM, _NT_HI * wid, _NT_LO * wid + _NREM)
    roff = jnp.where(wid < _NREM, 0, 1)
    pltpu.sync_copy(i1_hbm.at[pl.ds((start - roff) * _CHUNK, _NT_HI * _CHUNK)], i1all)
    pltpu.sync_copy(i2_hbm.at[pl.ds((start - roff) * _CHUNK, _NT_HI * _CHUNK)], i2all)

    def chunk_base(t):
        return (start + t) * _CHUNK

    def inputs_start(t, b):
        base = chunk_base(t)
        off = (roff + t) * _CHUNK
        for h in range(2):
            for k in range(8):
                i1_v[b, h, k * 16:(k + 1) * 16] = i1all[pl.ds(off + h * 128 + k * 16, 16)]
                i2_v[b, h, k * 16:(k + 1) * 16] = i2all[pl.ds(off + h * 128 + k * 16, 16)]
        for h in range(2):
            pltpu.async_copy(a_hbm.at[i1_v.at[b, h]], ga.at[b, pl.ds(h * 128, 128)],
                             sga.at[b])
            pltpu.async_copy(b_hbm.at[i2_v.at[b, h]], gb.at[b, pl.ds(h * 128, 128)],
                             sgb.at[b])
        pltpu.async_copy(bin_hbm.at[pl.ds(base, _CHUNK), pl.ds(0, 16)],
                         bin_v.at[b], sbi.at[b])

    def inputs_wait(b):
        for h in range(2):
            pltpu.make_async_copy(a_hbm.at[i1_v.at[b, h]],
                                  ga.at[b, pl.ds(h * 128, 128)], sga.at[b]).wait()
            pltpu.make_async_copy(b_hbm.at[i2_v.at[b, h]],
                                  gb.at[b, pl.ds(h * 128, 128)], sgb.at[b]).wait()
        pltpu.make_async_copy(bin_hbm.at[pl.ds(0, _CHUNK), pl.ds(0, 16)],
                              bin_v.at[b], sbi.at[b]).wait()

    def outputs_start(t, b):
        base = chunk_base(t)
        for h in range(2):
            pltpu.async_copy(ga.at[b, pl.ds(h * 128, 128)],
                             acce_s.at[i1_v.at[b, h]], sca.at[b], add=True)
            pltpu.async_copy(gb.at[b, pl.ds(h * 128, 128)],
                             acco_s.at[i2_v.at[b, h]], scb.at[b], add=True)
        pltpu.async_copy(bin_v.at[b], bout_hbm.at[pl.ds(base, _CHUNK), pl.ds(0, 16)],
                         sbo.at[b])

    def outputs_wait(b):
        for h in range(2):
            pltpu.make_async_copy(ga.at[b, pl.ds(h * 128, 128)],
                                  acce_s.at[i1_v.at[b, h]], sca.at[b]).wait()
            pltpu.make_async_copy(gb.at[b, pl.ds(h * 128, 128)],
                                  acco_s.at[i2_v.at[b, h]], scb.at[b]).wait()
        pltpu.make_async_copy(bin_v.at[b], bout_hbm.at[pl.ds(0, _CHUNK), pl.ds(0, 16)],
                              sbo.at[b]).wait()

    inputs_start(0, 0)

    def chunk(t, _):
        b = t & 1
        nb = 1 - b
        have_next = (t + 1) < n_t

        @pl.when((t >= 1) & have_next)
        def _():
            outputs_wait(nb)

        @pl.when(have_next)
        def _():
            inputs_start(t + 1, nb)

        inputs_wait(b)

        @plsc.parallel_loop(0, _CHUNK, 1, unroll=8)
        def edge(ei):
            ea0 = ga[b, ei, 0:16]
            ea1 = ga[b, ei, 16:32]
            eb0 = gb[b, ei, 0:16]
            eb1 = gb[b, ei, 16:32]
            c = bin_v[b, ei, 0:16]
            ec = jnp.exp(c)
            inv0 = 1.0 / (ea0 + eb0 + ec)
            inv1 = 1.0 / (ea1 + eb1 + ec)
            ga[b, ei, 0:16] = -(wlo * ea0) * inv0
            ga[b, ei, 16:32] = -(whi * ea1) * inv1
            gb[b, ei, 0:16] = (wlo * eb0) * inv0
            gb[b, ei, 16:32] = (whi * eb1) * inv1
            bin_v[b, ei, 0:16] = c + (wlo * ec) * inv0 + (whi * ec) * inv1

        outputs_start(t, b)
        return 0

    lax.fori_loop(0, n_t, chunk, 0)
    outputs_wait(0)
    outputs_wait(1)
    plsc.subcore_barrier()

    rbase = sid * _ROWS_PER_TILE
    obase = cid * _N_PAD + sid * _ROWS_PER_TILE
    pltpu.sync_copy(acce_s.at[pl.ds(rbase, _ROWS_PER_TILE)],
                    acce_hbm.at[pl.ds(obase, _ROWS_PER_TILE)])
    pltpu.sync_copy(acco_s.at[pl.ds(rbase, _ROWS_PER_TILE)],
                    acco_hbm.at[pl.ds(obase, _ROWS_PER_TILE)])


def _stage_b(a_tab, b_tab, binary, index1, index2, bcw):
    mesh = plsc.VectorSubcoreMesh(core_axis_name="c", subcore_axis_name="s",
                                  num_cores=_NC, num_subcores=_NS)
    fn = pl.kernel(
        _sc_body,
        out_type=(
            jax.ShapeDtypeStruct((_NC * _N_PAD, 32), jnp.float32),
            jax.ShapeDtypeStruct((_NC * _N_PAD, 32), jnp.float32),
            jax.ShapeDtypeStruct((_N_EDGES, 128), jnp.float32),
        ),
        mesh=mesh,
        compiler_params=pltpu.CompilerParams(use_tc_tiling_on_sc=False),
        scratch_types=[
            pltpu.VMEM((2, 2, 128), jnp.int32),
            pltpu.VMEM((2, 2, 128), jnp.int32),
            pltpu.VMEM((_NT_HI * _CHUNK,), jnp.int32),
            pltpu.VMEM((_NT_HI * _CHUNK,), jnp.int32),
            pltpu.VMEM((2, _CHUNK, 32), jnp.float32),
            pltpu.VMEM((2, _CHUNK, 32), jnp.float32),
            pltpu.VMEM((2, _CHUNK, 16), jnp.float32),
            pltpu.VMEM((32,), jnp.float32),
            pltpu.VMEM((_ROWS_PER_TILE, 32), jnp.float32),
            pltpu.VMEM_SHARED((_N_PAD, 32), jnp.float32),
            pltpu.VMEM_SHARED((_N_PAD, 32), jnp.float32),
            pltpu.SemaphoreType.DMA((2,)),
            pltpu.SemaphoreType.DMA((2,)),
            pltpu.SemaphoreType.DMA((2,)),
            pltpu.SemaphoreType.DMA((2,)),
            pltpu.SemaphoreType.DMA((2,)),
            pltpu.SemaphoreType.DMA((2,)),
        ],
    )
    return fn(a_tab, b_tab, binary, index1, index2, bcw)


def _final_body(u_ref, e0_ref, e1_ref, o0_ref, o1_ref, pe_ref, po_ref, out_ref):
    es = e0_ref[...] + e1_ref[...]
    osum = o0_ref[...] + o1_ref[...]
    out_ref[...] = (u_ref[...]
                    + jnp.dot(es, pe_ref[...], preferred_element_type=jnp.float32)
                    + jnp.dot(osum, po_ref[...], preferred_element_type=jnp.float32))


def _stage_c(u, acce, acco):
    spec32 = pl.BlockSpec((_BLK, 32), lambda i: (i, 0))
    const_spec = pl.BlockSpec((32, 64), lambda i: (0, 0))
    e0, e1 = acce[0:_N_NODES], acce[_N_PAD:_N_PAD + _N_NODES]
    o0, o1 = acco[0:_N_NODES], acco[_N_PAD:_N_PAD + _N_NODES]
    return pl.pallas_call(
        _final_body,
        grid=(_GRID,),
        in_specs=[pl.BlockSpec((_BLK, 64), lambda i: (i, 0)),
                  spec32, spec32, spec32, spec32, const_spec, const_spec],
        out_specs=pl.BlockSpec((_BLK, 64), lambda i: (i, 0)),
        out_shape=jax.ShapeDtypeStruct((_N_NODES, 64), jnp.float32),
    )(u, e0, e1, o0, o1, jnp.asarray(_PE), jnp.asarray(_PO))


def kernel(unary, binary, index1, index2, unary_clause_weights, binary_clause_weights):
    wcol = jnp.concatenate([jnp.repeat(unary_clause_weights, 3),
                            jnp.zeros((16,), jnp.float32)]).reshape(1, 64)
    u, a_tab, b_tab = _stage_a(unary, wcol)
    binp = jnp.dot(binary, jnp.asarray(_PADW), preferred_element_type=jnp.float32)
    acce, acco, boutp = _stage_b(a_tab, b_tab, binp, index1, index2,
                                 binary_clause_weights)
    out1 = _stage_c(u, acce, acco)
    out2 = boutp[:, 0:_NB]
    return (out1, out2)


# speedup vs baseline: 1.1784x; 1.1784x over previous
import functools
import numpy as np
import jax
import jax.numpy as jnp
from jax import lax
from jax.experimental import pallas as pl
from jax.experimental.pallas import tpu as pltpu
from jax.experimental.pallas import tpu_sc as plsc

_N_NODES = 10000
_N_EDGES = 320000
_NU = 64
_NB = 16

_NC = 2
_NS = 16
_NW = _NC * _NS
_CHUNK = 256
_NCHUNKS = _N_EDGES // _CHUNK
_NT_HI = (_NCHUNKS + _NW - 1) // _NW
_NT_LO = _NCHUNKS // _NW
_NREM = _NCHUNKS - _NT_LO * _NW
_N_PAD = 10240
_ROWS_PER_TILE = _N_PAD // _NS

_BLK = 1000
_GRID = _N_NODES // _BLK

_SGN = np.ones((1, 64), np.float32)
for _i in range(16):
    _SGN[0, 3 * _i + 1] = -1.0
_MASK = np.zeros((1, 64), np.float32)
_MASK[0, 48:] = -1e30
_G = np.zeros((64, 64), np.float32)
for _i in range(16):
    _G[3 * _i:3 * _i + 3, 3 * _i:3 * _i + 3] = 1.0
_ESELN = np.zeros((64, 32), np.float32)
_OSEL = np.zeros((64, 32), np.float32)
_PE = np.zeros((32, 64), np.float32)
_PO = np.zeros((32, 64), np.float32)
_PADW = np.zeros((16, 128), np.float32)
for _j in range(16):
    _PADW[_j, _j] = 1.0
for _j in range(32):
    _ESELN[2 * _j, _j] = -1.0
    _OSEL[2 * _j + 1, _j] = 1.0
    _PE[_j, 2 * _j] = 1.0
    _PO[_j, 2 * _j + 1] = 1.0


def _unary_body(x_ref, w_ref, sgn_ref, msk_ref, g_ref, en_ref, os_ref,
                u_ref, a_ref, b_ref):
    x = x_ref[...]
    cm = x * sgn_ref[...] + msk_ref[...]
    m = jnp.max(cm, axis=1, keepdims=True)
    e = jnp.exp(cm - m)
    s = jnp.dot(e, g_ref[...], preferred_element_type=jnp.float32)
    r = e / (s + 1e-30)
    u = x + sgn_ref[...] * r * w_ref[...]
    u_ref[...] = u
    a_ref[...] = jnp.exp(jnp.dot(u, en_ref[...], preferred_element_type=jnp.float32))
    b_ref[...] = jnp.exp(jnp.dot(u, os_ref[...], preferred_element_type=jnp.float32))


def _stage_a(unary, wcol):
    const_spec = lambda shape: pl.BlockSpec(shape, lambda i: (0, 0))
    return pl.pallas_call(
        _unary_body,
        grid=(_GRID,),
        in_specs=[
            pl.BlockSpec((_BLK, 64), lambda i: (i, 0)),
            const_spec((1, 64)), const_spec((1, 64)), const_spec((1, 64)),
            const_spec((64, 64)), const_spec((64, 32)), const_spec((64, 32)),
        ],
        out_specs=[
            pl.BlockSpec((_BLK, 64), lambda i: (i, 0)),
            pl.BlockSpec((_BLK, 32), lambda i: (i, 0)),
            pl.BlockSpec((_BLK, 32), lambda i: (i, 0)),
        ],
        out_shape=[
            jax.ShapeDtypeStruct((_N_NODES, 64), jnp.float32),
            jax.ShapeDtypeStruct((_N_NODES, 32), jnp.float32),
            jax.ShapeDtypeStruct((_N_NODES, 32), jnp.float32),
        ],
    )(unary, wcol, jnp.asarray(_SGN), jnp.asarray(_MASK), jnp.asarray(_G),
      jnp.asarray(_ESELN), jnp.asarray(_OSEL))


def _sc_body(a_hbm, b_hbm, bin_hbm, i1_hbm, i2_hbm, w_hbm,
             acce_hbm, acco_hbm, bout_hbm,
             i1_v, i2_v, i1all, i2all, ga, gb, bin_v, w_v, zbuf,
             acce_s, acco_s,
             sga, sgb, sbi, sca, scb, sbo):
    cid = lax.axis_index("c")
    sid = lax.axis_index("s")
    wid = sid * _NC + cid

    zeros16 = jnp.zeros((16,), jnp.float32)

    def zrow(i, _):
        zbuf[i, 0:16] = zeros16
        zbuf[i, 16:32] = zeros16
        return 0

    lax.fori_loop(0, _ROWS_PER_TILE, zrow, 0)
    pltpu.sync_copy(zbuf, acce_s.at[pl.ds(sid * _ROWS_PER_TILE, _ROWS_PER_TILE)])
    pltpu.sync_copy(zbuf, acco_s.at[pl.ds(sid * _ROWS_PER_TILE, _ROWS_PER_TILE)])
    pltpu.sync_copy(w_hbm, w_v)
    plsc.subcore_barrier()

    wlo = w_v[0:16]
    whi = w_v[16:32]

    n_t = jnp.where(wid < _NREM, _NT_HI, _NT_LO)
    start = jnp.where(wid < _NREM, _NT_HI * wid, _NT_LO * wid + _NREM)
    roff = jnp.where(wid < _NREM, 0, 1)
    pltpu.sync_copy(i1_hbm.at[pl.ds((start - roff) * _CHUNK, _NT_HI * _CHUNK)], i1all)
    pltpu.sync_copy(i2_hbm.at[pl.ds((start - roff) * _CHUNK, _NT_HI * _CHUNK)], i2all)

    def chunk_base(t):
        return (start + t) * _CHUNK

    def inputs_start(t, b):
        base = chunk_base(t)
        off = (roff + t) * _CHUNK
        for h in range(2):
            for k in range(8):
                i1_v[b, h, k * 16:(k + 1) * 16] = i1all[pl.ds(off + h * 128 + k * 16, 16)]
                i2_v[b, h, k * 16:(k + 1) * 16] = i2all[pl.ds(off + h * 128 + k * 16, 16)]
        for h in range(2):
            pltpu.async_copy(a_hbm.at[i1_v.at[b, h]], ga.at[b, pl.ds(h * 128, 128)],
                             sga.at[b])
            pltpu.async_copy(b_hbm.at[i2_v.at[b, h]], gb.at[b, pl.ds(h * 128, 128)],
                             sgb.at[b])
        pltpu.async_copy(bin_hbm.at[pl.ds(base, _CHUNK), pl.ds(0, 16)],
                         bin_v.at[b], sbi.at[b])

    def inputs_wait(b):
        for h in range(2):
            pltpu.make_async_copy(a_hbm.at[i1_v.at[b, h]],
                                  ga.at[b, pl.ds(h * 128, 128)], sga.at[b]).wait()
            pltpu.make_async_copy(b_hbm.at[i2_v.at[b, h]],
                                  gb.at[b, pl.ds(h * 128, 128)], sgb.at[b]).wait()
        pltpu.make_async_copy(bin_hbm.at[pl.ds(0, _CHUNK), pl.ds(0, 16)],
                              bin_v.at[b], sbi.at[b]).wait()

    def outputs_start(t, b):
        base = chunk_base(t)
        for h in range(2):
            pltpu.async_copy(ga.at[b, pl.ds(h * 128, 128)],
                             acce_s.at[i1_v.at[b, h]], sca.at[b], add=True)
            pltpu.async_copy(gb.at[b, pl.ds(h * 128, 128)],
                             acco_s.at[i2_v.at[b, h]], scb.at[b], add=True)
        pltpu.async_copy(bin_v.at[b], bout_hbm.at[pl.ds(base, _CHUNK), pl.ds(0, 16)],
                         sbo.at[b])

    def outputs_wait(b):
        for h in range(2):
            pltpu.make_async_copy(ga.at[b, pl.ds(h * 128, 128)],
                                  acce_s.at[i1_v.at[b, h]], sca.at[b]).wait()
            pltpu.make_async_copy(gb.at[b, pl.ds(h * 128, 128)],
                                  acco_s.at[i2_v.at[b, h]], scb.at[b]).wait()
        pltpu.make_async_copy(bin_v.at[b], bout_hbm.at[pl.ds(0, _CHUNK), pl.ds(0, 16)],
                              sbo.at[b]).wait()

    inputs_start(0, 0)

    def chunk(t, _):
        b = t & 1
        nb = 1 - b
        have_next = (t + 1) < n_t

        @pl.when((t >= 1) & have_next)
        def _():
            outputs_wait(nb)

        @pl.when(have_next)
        def _():
            inputs_start(t + 1, nb)

        inputs_wait(b)

        @plsc.parallel_loop(0, _CHUNK, 1, unroll=4)
        def edge(ei):
            ea0 = ga[b, ei, 0:16]
            ea1 = ga[b, ei, 16:32]
            eb0 = gb[b, ei, 0:16]
            eb1 = gb[b, ei, 16:32]
            c = bin_v[b, ei, 0:16]
            ec = jnp.exp(c)
            inv0 = 1.0 / (ea0 + eb0 + ec)
            inv1 = 1.0 / (ea1 + eb1 + ec)
            ga[b, ei, 0:16] = -(wlo * ea0) * inv0
            ga[b, ei, 16:32] = -(whi * ea1) * inv1
            gb[b, ei, 0:16] = (wlo * eb0) * inv0
            gb[b, ei, 16:32] = (whi * eb1) * inv1
            bin_v[b, ei, 0:16] = c + (wlo * ec) * inv0 + (whi * ec) * inv1

        outputs_start(t, b)
        return 0

    lax.fori_loop(0, n_t, chunk, 0)
    outputs_wait(0)
    outputs_wait(1)
    plsc.subcore_barrier()

    rbase = sid * _ROWS_PER_TILE
    obase = cid * _N_PAD + sid * _ROWS_PER_TILE
    pltpu.sync_copy(acce_s.at[pl.ds(rbase, _ROWS_PER_TILE)],
                    acce_hbm.at[pl.ds(obase, _ROWS_PER_TILE)])
    pltpu.sync_copy(acco_s.at[pl.ds(rbase, _ROWS_PER_TILE)],
                    acco_hbm.at[pl.ds(obase, _ROWS_PER_TILE)])


def _stage_b(a_tab, b_tab, binary, index1, index2, bcw):
    mesh = plsc.VectorSubcoreMesh(core_axis_name="c", subcore_axis_name="s",
                                  num_cores=_NC, num_subcores=_NS)
    fn = pl.kernel(
        _sc_body,
        out_type=(
            jax.ShapeDtypeStruct((_NC * _N_PAD, 32), jnp.float32),
            jax.ShapeDtypeStruct((_NC * _N_PAD, 32), jnp.float32),
            jax.ShapeDtypeStruct((_N_EDGES, 128), jnp.float32),
        ),
        mesh=mesh,
        compiler_params=pltpu.CompilerParams(use_tc_tiling_on_sc=False),
        scratch_types=[
            pltpu.VMEM((2, 2, 128), jnp.int32),
            pltpu.VMEM((2, 2, 128), jnp.int32),
            pltpu.VMEM((_NT_HI * _CHUNK,), jnp.int32),
            pltpu.VMEM((_NT_HI * _CHUNK,), jnp.int32),
            pltpu.VMEM((2, _CHUNK, 32), jnp.float32),
            pltpu.VMEM((2, _CHUNK, 32), jnp.float32),
            pltpu.VMEM((2, _CHUNK, 16), jnp.float32),
            pltpu.VMEM((32,), jnp.float32),
            pltpu.VMEM((_ROWS_PER_TILE, 32), jnp.float32),
            pltpu.VMEM_SHARED((_N_PAD, 32), jnp.float32),
            pltpu.VMEM_SHARED((_N_PAD, 32), jnp.float32),
            pltpu.SemaphoreType.DMA((2,)),
            pltpu.SemaphoreType.DMA((2,)),
            pltpu.SemaphoreType.DMA((2,)),
            pltpu.SemaphoreType.DMA((2,)),
            pltpu.SemaphoreType.DMA((2,)),
            pltpu.SemaphoreType.DMA((2,)),
        ],
    )
    return fn(a_tab, b_tab, binary, index1, index2, bcw)


def _final_body(u_ref, e0_ref, e1_ref, o0_ref, o1_ref, pe_ref, po_ref, out_ref):
    es = e0_ref[...] + e1_ref[...]
    osum = o0_ref[...] + o1_ref[...]
    out_ref[...] = (u_ref[...]
                    + jnp.dot(es, pe_ref[...], preferred_element_type=jnp.float32)
                    + jnp.dot(osum, po_ref[...], preferred_element_type=jnp.float32))


def _stage_c(u, acce, acco):
    spec32 = pl.BlockSpec((_BLK, 32), lambda i: (i, 0))
    const_spec = pl.BlockSpec((32, 64), lambda i: (0, 0))
    e0, e1 = acce[0:_N_NODES], acce[_N_PAD:_N_PAD + _N_NODES]
    o0, o1 = acco[0:_N_NODES], acco[_N_PAD:_N_PAD + _N_NODES]
    return pl.pallas_call(
        _final_body,
        grid=(_GRID,),
        in_specs=[pl.BlockSpec((_BLK, 64), lambda i: (i, 0)),
                  spec32, spec32, spec32, spec32, const_spec, const_spec],
        out_specs=pl.BlockSpec((_BLK, 64), lambda i: (i, 0)),
        out_shape=jax.ShapeDtypeStruct((_N_NODES, 64), jnp.float32),
    )(u, e0, e1, o0, o1, jnp.asarray(_PE), jnp.asarray(_PO))


def kernel(unary, binary, index1, index2, unary_clause_weights, binary_clause_weights):
    wcol = jnp.concatenate([jnp.repeat(unary_clause_weights, 3),
                            jnp.zeros((16,), jnp.float32)]).reshape(1, 64)
    u, a_tab, b_tab = _stage_a(unary, wcol)
    binp = jnp.dot(binary, jnp.asarray(_PADW), preferred_element_type=jnp.float32)
    acce, acco, boutp = _stage_b(a_tab, b_tab, binp, index1, index2,
                                 binary_clause_weights)
    out1 = _stage_c(u, acce, acco)
    out2 = boutp[:, 0:_NB]
    return (out1, out2)


# speedup vs baseline: 1.1788x; 1.0004x over previous
import numpy as np
import jax
import jax.numpy as jnp
from jax import lax
from jax.experimental import pallas as pl
from jax.experimental.pallas import tpu as pltpu
from jax.experimental.pallas import tpu_sc as plsc

_N_NODES = 10000
_N_EDGES = 320000
_NU = 64
_NB = 16

_NC = 2
_NS = 16
_NW = _NC * _NS
_CHUNK = 256
_NCHUNKS = _N_EDGES // _CHUNK
_NT_HI = (_NCHUNKS + _NW - 1) // _NW
_NT_LO = _NCHUNKS // _NW
_NREM = _NCHUNKS - _NT_LO * _NW
_N_PAD = 10240
_ROWS_PER_TILE = _N_PAD // _NS

_BLK = 1000
_GRID = _N_NODES // _BLK

_SGN = np.ones((1, 64), np.float32)
for _i in range(16):
    _SGN[0, 3 * _i + 1] = -1.0
_MASK = np.zeros((1, 64), np.float32)
_MASK[0, 48:] = -1e30
_G = np.zeros((64, 64), np.float32)
for _i in range(16):
    _G[3 * _i:3 * _i + 3, 3 * _i:3 * _i + 3] = 1.0
_ESELN = np.zeros((64, 32), np.float32)
_OSEL = np.zeros((64, 32), np.float32)
_PE = np.zeros((32, 64), np.float32)
_PO = np.zeros((32, 64), np.float32)
_PADW = np.zeros((16, 128), np.float32)
for _j in range(16):
    _PADW[_j, _j] = 1.0
for _j in range(32):
    _ESELN[2 * _j, _j] = -1.0
    _OSEL[2 * _j + 1, _j] = 1.0
    _PE[_j, 2 * _j] = 1.0
    _PO[_j, 2 * _j + 1] = 1.0


def _unary_body(x_ref, w_ref, sgn_ref, msk_ref, g_ref, en_ref, os_ref,
                u_ref, a_ref, b_ref):
    x = x_ref[...]
    cm = x * sgn_ref[...] + msk_ref[...]
    m = jnp.max(cm, axis=1, keepdims=True)
    e = jnp.exp(cm - m)
    s = jnp.dot(e, g_ref[...], preferred_element_type=jnp.float32)
    r = e / (s + 1e-30)
    u = x + sgn_ref[...] * r * w_ref[...]
    u_ref[...] = u
    a_ref[...] = jnp.exp(jnp.dot(u, en_ref[...], preferred_element_type=jnp.float32))
    b_ref[...] = jnp.exp(jnp.dot(u, os_ref[...], preferred_element_type=jnp.float32))


def _stage_a(unary, wcol):
    const_spec = lambda shape: pl.BlockSpec(shape, lambda i: (0, 0))
    return pl.pallas_call(
        _unary_body,
        grid=(_GRID,),
        in_specs=[
            pl.BlockSpec((_BLK, 64), lambda i: (i, 0)),
            const_spec((1, 64)), const_spec((1, 64)), const_spec((1, 64)),
            const_spec((64, 64)), const_spec((64, 32)), const_spec((64, 32)),
        ],
        out_specs=[
            pl.BlockSpec((_BLK, 64), lambda i: (i, 0)),
            pl.BlockSpec((_BLK, 32), lambda i: (i, 0)),
            pl.BlockSpec((_BLK, 32), lambda i: (i, 0)),
        ],
        out_shape=[
            jax.ShapeDtypeStruct((_N_NODES, 64), jnp.float32),
            jax.ShapeDtypeStruct((_N_NODES, 32), jnp.float32),
            jax.ShapeDtypeStruct((_N_NODES, 32), jnp.float32),
        ],
    )(unary, wcol, jnp.asarray(_SGN), jnp.asarray(_MASK), jnp.asarray(_G),
      jnp.asarray(_ESELN), jnp.asarray(_OSEL))


def _sc_body(a_hbm, b_hbm, bin_hbm, i1_hbm, i2_hbm, w_hbm,
             acce_hbm, acco_hbm, bout_hbm,
             i1_v, i2_v, i1all, i2all, ga, gb, bin_v, w_v, zbuf,
             acce_s, acco_s,
             sga, sgb, sbi, sca, scb, sbo):
    cid = lax.axis_index("c")
    sid = lax.axis_index("s")
    wid = sid * _NC + cid

    zeros16 = jnp.zeros((16,), jnp.float32)

    def zrow(i, _):
        zbuf[i, 0:16] = zeros16
        zbuf[i, 16:32] = zeros16
        return 0

    lax.fori_loop(0, _ROWS_PER_TILE, zrow, 0)
    pltpu.sync_copy(zbuf, acce_s.at[pl.ds(sid * _ROWS_PER_TILE, _ROWS_PER_TILE)])
    pltpu.sync_copy(zbuf, acco_s.at[pl.ds(sid * _ROWS_PER_TILE, _ROWS_PER_TILE)])
    pltpu.sync_copy(w_hbm, w_v)
    plsc.subcore_barrier()

    wlo = w_v[0:16]
    whi = w_v[16:32]

    n_t = jnp.where(wid < _NREM, _NT_HI, _NT_LO)
    start = jnp.where(wid < _NREM, _NT_HI * wid, _NT_LO * wid + _NREM)
    roff = jnp.where(wid < _NREM, 0, 1)
    pltpu.sync_copy(i1_hbm.at[pl.ds((start - roff) * _CHUNK, _NT_HI * _CHUNK)], i1all)
    pltpu.sync_copy(i2_hbm.at[pl.ds((start - roff) * _CHUNK, _NT_HI * _CHUNK)], i2all)

    def chunk_base(t):
        return (start + t) * _CHUNK

    def inputs_start(t, b):
        base = chunk_base(t)
        off = (roff + t) * _CHUNK
        for h in range(2):
            for k in range(8):
                i1_v[b, h, k * 16:(k + 1) * 16] = i1all[pl.ds(off + h * 128 + k * 16, 16)]
                i2_v[b, h, k * 16:(k + 1) * 16] = i2all[pl.ds(off + h * 128 + k * 16, 16)]
        for h in range(2):
            pltpu.async_copy(a_hbm.at[i1_v.at[b, h]], ga.at[b, pl.ds(h * 128, 128)],
                             sga.at[b])
            pltpu.async_copy(b_hbm.at[i2_v.at[b, h]], gb.at[b, pl.ds(h * 128, 128)],
                             sgb.at[b])
        pltpu.async_copy(bin_hbm.at[pl.ds(base, _CHUNK), pl.ds(0, 16)],
                         bin_v.at[b], sbi.at[b])

    def inputs_wait(b):
        for h in range(2):
            pltpu.make_async_copy(a_hbm.at[i1_v.at[b, h]],
                                  ga.at[b, pl.ds(h * 128, 128)], sga.at[b]).wait()
            pltpu.make_async_copy(b_hbm.at[i2_v.at[b, h]],
                                  gb.at[b, pl.ds(h * 128, 128)], sgb.at[b]).wait()
        pltpu.make_async_copy(bin_hbm.at[pl.ds(0, _CHUNK), pl.ds(0, 16)],
                              bin_v.at[b], sbi.at[b]).wait()

    def outputs_start(t, b):
        base = chunk_base(t)
        for h in range(2):
            pltpu.async_copy(ga.at[b, pl.ds(h * 128, 128)],
                             acce_s.at[i1_v.at[b, h]], sca.at[b], add=True)
            pltpu.async_copy(gb.at[b, pl.ds(h * 128, 128)],
                             acco_s.at[i2_v.at[b, h]], scb.at[b], add=True)
        pltpu.async_copy(bin_v.at[b], bout_hbm.at[pl.ds(base, _CHUNK), pl.ds(0, 16)],
                         sbo.at[b])

    def outputs_wait(b):
        for h in range(2):
            pltpu.make_async_copy(ga.at[b, pl.ds(h * 128, 128)],
                                  acce_s.at[i1_v.at[b, h]], sca.at[b]).wait()
            pltpu.make_async_copy(gb.at[b, pl.ds(h * 128, 128)],
                                  acco_s.at[i2_v.at[b, h]], scb.at[b]).wait()
        pltpu.make_async_copy(bin_v.at[b], bout_hbm.at[pl.ds(0, _CHUNK), pl.ds(0, 16)],
                              sbo.at[b]).wait()

    inputs_start(0, 0)

    def chunk(t, _):
        b = t & 1
        nb = 1 - b
        have_next = (t + 1) < n_t

        @pl.when((t >= 1) & have_next)
        def _():
            outputs_wait(nb)

        @pl.when(have_next)
        def _():
            inputs_start(t + 1, nb)

        inputs_wait(b)

        @plsc.parallel_loop(0, _CHUNK, 1, unroll=4)
        def edge(ei):
            ea0 = ga[b, ei, 0:16]
            ea1 = ga[b, ei, 16:32]
            eb0 = gb[b, ei, 0:16]
            eb1 = gb[b, ei, 16:32]
            c = bin_v[b, ei, 0:16]
            ec = jnp.exp(c)
            inv0 = 1.0 / (ea0 + eb0 + ec)
            inv1 = 1.0 / (ea1 + eb1 + ec)
            ga[b, ei, 0:16] = -(wlo * ea0) * inv0
            ga[b, ei, 16:32] = -(whi * ea1) * inv1
            gb[b, ei, 0:16] = (wlo * eb0) * inv0
            gb[b, ei, 16:32] = (whi * eb1) * inv1
            bin_v[b, ei, 0:16] = c + (wlo * ec) * inv0 + (whi * ec) * inv1

        outputs_start(t, b)
        return 0

    lax.fori_loop(0, n_t, chunk, 0)
    outputs_wait(0)
    outputs_wait(1)
    plsc.subcore_barrier()

    rbase = sid * _ROWS_PER_TILE
    obase = cid * _N_PAD + sid * _ROWS_PER_TILE
    pltpu.sync_copy(acce_s.at[pl.ds(rbase, _ROWS_PER_TILE)],
                    acce_hbm.at[pl.ds(obase, _ROWS_PER_TILE)])
    pltpu.sync_copy(acco_s.at[pl.ds(rbase, _ROWS_PER_TILE)],
                    acco_hbm.at[pl.ds(obase, _ROWS_PER_TILE)])


def _stage_b(a_tab, b_tab, binary, index1, index2, bcw):
    mesh = plsc.VectorSubcoreMesh(core_axis_name="c", subcore_axis_name="s",
                                  num_cores=_NC, num_subcores=_NS)
    fn = pl.kernel(
        _sc_body,
        out_type=(
            jax.ShapeDtypeStruct((_NC * _N_PAD, 32), jnp.float32),
            jax.ShapeDtypeStruct((_NC * _N_PAD, 32), jnp.float32),
            jax.ShapeDtypeStruct((_N_EDGES, 128), jnp.float32),
        ),
        mesh=mesh,
        compiler_params=pltpu.CompilerParams(use_tc_tiling_on_sc=False),
        scratch_types=[
            pltpu.VMEM((2, 2, 128), jnp.int32),
            pltpu.VMEM((2, 2, 128), jnp.int32),
            pltpu.VMEM((_NT_HI * _CHUNK,), jnp.int32),
            pltpu.VMEM((_NT_HI * _CHUNK,), jnp.int32),
            pltpu.VMEM((2, _CHUNK, 32), jnp.float32),
            pltpu.VMEM((2, _CHUNK, 32), jnp.float32),
            pltpu.VMEM((2, _CHUNK, 16), jnp.float32),
            pltpu.VMEM((32,), jnp.float32),
            pltpu.VMEM((_ROWS_PER_TILE, 32), jnp.float32),
            pltpu.VMEM_SHARED((_N_PAD, 32), jnp.float32),
            pltpu.VMEM_SHARED((_N_PAD, 32), jnp.float32),
            pltpu.SemaphoreType.DMA((2,)),
            pltpu.SemaphoreType.DMA((2,)),
            pltpu.SemaphoreType.DMA((2,)),
            pltpu.SemaphoreType.DMA((2,)),
            pltpu.SemaphoreType.DMA((2,)),
            pltpu.SemaphoreType.DMA((2,)),
        ],
    )
    return fn(a_tab, b_tab, binary, index1, index2, bcw)


def _final_body(u_ref, e0_ref, e1_ref, o0_ref, o1_ref, pe_ref, po_ref, out_ref):
    es = e0_ref[...] + e1_ref[...]
    osum = o0_ref[...] + o1_ref[...]
    out_ref[...] = (u_ref[...]
                    + jnp.dot(es, pe_ref[...], preferred_element_type=jnp.float32)
                    + jnp.dot(osum, po_ref[...], preferred_element_type=jnp.float32))


def _stage_c(u, acce, acco):
    spec32 = pl.BlockSpec((_BLK, 32), lambda i: (i, 0))
    const_spec = pl.BlockSpec((32, 64), lambda i: (0, 0))
    e0, e1 = acce[0:_N_NODES], acce[_N_PAD:_N_PAD + _N_NODES]
    o0, o1 = acco[0:_N_NODES], acco[_N_PAD:_N_PAD + _N_NODES]
    return pl.pallas_call(
        _final_body,
        grid=(_GRID,),
        in_specs=[pl.BlockSpec((_BLK, 64), lambda i: (i, 0)),
                  spec32, spec32, spec32, spec32, const_spec, const_spec],
        out_specs=pl.BlockSpec((_BLK, 64), lambda i: (i, 0)),
        out_shape=jax.ShapeDtypeStruct((_N_NODES, 64), jnp.float32),
    )(u, e0, e1, o0, o1, jnp.asarray(_PE), jnp.asarray(_PO))


def kernel(unary, binary, index1, index2, unary_clause_weights, binary_clause_weights):
    wcol = jnp.concatenate([jnp.repeat(unary_clause_weights, 3),
                            jnp.zeros((16,), jnp.float32)]).reshape(1, 64)
    u, a_tab, b_tab = _stage_a(unary, wcol)
    binp = jnp.dot(binary, jnp.asarray(_PADW), preferred_element_type=jnp.float32)
    acce, acco, boutp = _stage_b(a_tab, b_tab, binp, index1, index2,
                                 binary_clause_weights)
    out1 = _stage_c(u, acce, acco)
    out2 = boutp[:, 0:_NB]
    return (out1, out2)
